# baseline (speedup 1.0000x reference)
import numpy as np
import jax
import jax.numpy as jnp
from jax import lax
from jax.experimental import pallas as pl
from jax.experimental.pallas import tpu as pltpu
from jax.experimental.pallas import tpu_sc as plsc

N = 10000
E = 160000
D = 256
G = 64
EPS = 1e-5

NC = 2
NS = 16
HALF = D // NC
W = 128
WD = 256
WPS = 80
WPD = 20
NWIN_P = NS * WPS
E_PAD = NWIN_P * W
NP = 10240
STRIPE = NP // NS

BLK = 400
NBLK = N // BLK

_mesh = plsc.VectorSubcoreMesh(
    core_axis_name="c", subcore_axis_name="s", num_cores=NC, num_subcores=NS
)

_CHUNKS = [(o, min(128, STRIPE - o)) for o in range(0, STRIPE, 128)]


def _fill_f32(ref, rows, cols, val):
    @pl.loop(0, rows)
    def _(i):
        for j in range(cols // 16):
            ref[i, pl.ds(j * 16, 16)] = jnp.full((16,), val, jnp.float32)


def _wait_dma(hbm_ref, vmem_ref, sem):
    pltpu.make_async_copy(hbm_ref.at[pl.ds(0, W)], vmem_ref, sem).wait()


def _segsum_body(t_hbm, src_hbm, dst_hbm, out_hbm, idx_s, idx_d, rows, acc):
    c = lax.axis_index("c")
    s = lax.axis_index("s")
    base = s * STRIPE

    _fill_f32(rows, W, HALF, 0.0)
    for o, sz in _CHUNKS:
        pltpu.sync_copy(rows.at[pl.ds(0, sz)], acc.at[pl.ds(base + o, sz)])
    plsc.subcore_barrier()

    row_off = c * NP

    @pl.loop(0, WPS)
    def _(i):
        w = s + i * NS
        off = w * W
        pltpu.sync_copy(src_hbm.at[pl.ds(off, W)], idx_s)
        pltpu.sync_copy(dst_hbm.at[pl.ds(off, W)], idx_d)
        for j in range(W // 16):
            sl = pl.ds(j * 16, 16)
            idx_s[sl] = idx_s[sl] + row_off
        pltpu.sync_copy(t_hbm.at[idx_s], rows)
        pltpu.sync_copy(rows, acc.at[idx_d], add=True)

    plsc.subcore_barrier()
    for o, sz in _CHUNKS:
        pltpu.sync_copy(
            acc.at[pl.ds(base + o, sz)],
            out_hbm.at[pl.ds(c * NP + base + o, sz)],
        )


_segsum = pl.kernel(
    _segsum_body,
    out_type=jax.ShapeDtypeStruct((NC * NP, HALF), jnp.float32),
    mesh=_mesh,
    scratch_types=[
        pltpu.VMEM((W,), jnp.int32),
        pltpu.VMEM((W,), jnp.int32),
        pltpu.VMEM((W, HALF), jnp.float32),
        pltpu.VMEM_SHARED((NP, HALF), jnp.float32),
    ],
)


def _deg_body(dst_hbm, out_hbm, idx_d, ones, accd):
    c = lax.axis_index("c")
    s = lax.axis_index("s")
    base = s * STRIPE

    _fill_f32(ones, WD, HALF, 0.0)
    for o, sz in _CHUNKS:
        pltpu.sync_copy(ones.at[pl.ds(0, sz)], accd.at[pl.ds(base + o, sz)])
    _fill_f32(ones, WD, HALF, 1.0)
    plsc.subcore_barrier()

    @pl.loop(0, WPD)
    def _(i):
        off = ((c * NS + s) * WPD + i) * WD
        pltpu.sync_copy(dst_hbm.at[pl.ds(off, WD)], idx_d)
        pltpu.sync_copy(ones, accd.at[idx_d], add=True)

    plsc.subcore_barrier()
    for o, sz in _CHUNKS:
        pltpu.sync_copy(
            accd.at[pl.ds(base + o, sz)],
            out_hbm.at[pl.ds(c * NP + base + o, sz)],
        )


_deg = pl.kernel(
    _deg_body,
    out_type=jax.ShapeDtypeStruct((NC * NP, HALF), jnp.float32),
    mesh=_mesh,
    scratch_types=[
        pltpu.VMEM((WD,), jnp.int32),
        pltpu.VMEM((WD, HALF), jnp.float32),
        pltpu.VMEM_SHARED((NP, HALF), jnp.float32),
    ],
)


def _mm0_body(x_ref, wl_ref, wr_ref, t_ref, u_ref):
    xb = x_ref[...]
    t = jnp.dot(xb, wl_ref[...], preferred_element_type=jnp.float32)
    t_ref[0] = t[:, :HALF]
    t_ref[1] = t[:, HALF:]
    u_ref[...] = jnp.dot(xb, wr_ref[...], preferred_element_type=jnp.float32)


def _mm0(x, wl, wr):
    return pl.pallas_call(
        _mm0_body,
        grid=(NBLK,),
        in_specs=[
            pl.BlockSpec((BLK, D), lambda i: (i, 0)),
            pl.BlockSpec((D, D), lambda i: (0, 0)),
            pl.BlockSpec((D, D), lambda i: (0, 0)),
        ],
        out_specs=[
            pl.BlockSpec((NC, BLK, HALF), lambda i: (0, i, 0)),
            pl.BlockSpec((BLK, D), lambda i: (i, 0)),
        ],
        out_shape=[
            jax.ShapeDtypeStruct((NC, NP, HALF), jnp.float32),
            jax.ShapeDtypeStruct((N, D), jnp.float32),
        ],
    )(x, wl, wr)


def _act(s_ref, u_ref, deg_ref, b_ref, g_ref, be_ref):
    sfull = jnp.concatenate([s_ref[0], s_ref[1]], axis=1)
    deg = deg_ref[0, :, 0:1] + deg_ref[1, :, 0:1]
    agg = sfull / jnp.maximum(deg, 1.0)
    h = agg + u_ref[...] + b_ref[...]
    h = g_ref[...] * (h * (1.0 / np.sqrt(1.0 + EPS))) + be_ref[...]
    return jnp.maximum(h, 0.0)


def _mid_body(s_ref, u_ref, deg_ref, b_ref, g_ref, be_ref, wl_ref, wr_ref,
              t_ref, u2_ref):
    h = _act(s_ref, u_ref, deg_ref, b_ref, g_ref, be_ref)
    t = jnp.dot(h, wl_ref[...], preferred_element_type=jnp.float32)
    t_ref[0] = t[:, :HALF]
    t_ref[1] = t[:, HALF:]
    u2_ref[...] = jnp.dot(h, wr_ref[...], preferred_element_type=jnp.float32)


def _mid(s_, u, deg2, b, g, be, wl, wr):
    return pl.pallas_call(
        _mid_body,
        grid=(NBLK,),
        in_specs=[
            pl.BlockSpec((NC, BLK, HALF), lambda i: (0, i, 0)),
            pl.BlockSpec((BLK, D), lambda i: (i, 0)),
            pl.BlockSpec((NC, BLK, HALF), lambda i: (0, i, 0)),
            pl.BlockSpec((1, D), lambda i: (0, 0)),
            pl.BlockSpec((1, D), lambda i: (0, 0)),
            pl.BlockSpec((1, D), lambda i: (0, 0)),
            pl.BlockSpec((D, D), lambda i: (0, 0)),
            pl.BlockSpec((D, D), lambda i: (0, 0)),
        ],
        out_specs=[
            pl.BlockSpec((NC, BLK, HALF), lambda i: (0, i, 0)),
            pl.BlockSpec((BLK, D), lambda i: (i, 0)),
        ],
        out_shape=[
            jax.ShapeDtypeStruct((NC, NP, HALF), jnp.float32),
            jax.ShapeDtypeStruct((N, D), jnp.float32),
        ],
    )(s_, u, deg2, b, g, be, wl, wr)


def _final_body(s_ref, u_ref, deg_ref, b_ref, g_ref, be_ref, batch_ref,
                out_ref, acc_ref, cnt_ref):
    i = pl.program_id(0)

    @pl.when(i == 0)
    def _():
        acc_ref[...] = jnp.zeros((G, D), jnp.float32)
        cnt_ref[...] = jnp.zeros((G, 1), jnp.float32)

    h = _act(s_ref, u_ref, deg_ref, b_ref, g_ref, be_ref)
    bt = batch_ref[0, 0, :]
    onehot = (bt[None, :] == lax.broadcasted_iota(jnp.int32, (G, BLK), 0))
    onehot = onehot.astype(jnp.float32)
    acc_ref[...] += jnp.dot(onehot, h, preferred_element_type=jnp.float32)
    cnt_ref[...] += jnp.sum(onehot, axis=1, keepdims=True)

    @pl.when(i == NBLK - 1)
    def _():
        out_ref[...] = acc_ref[...] / jnp.maximum(cnt_ref[...], 1.0)


def _final(s_, u, deg2, b, g, be, batch3):
    return pl.pallas_call(
        _final_body,
        grid=(NBLK,),
        in_specs=[
            pl.BlockSpec((NC, BLK, HALF), lambda i: (0, i, 0)),
            pl.BlockSpec((BLK, D), lambda i: (i, 0)),
            pl.BlockSpec((NC, BLK, HALF), lambda i: (0, i, 0)),
            pl.BlockSpec((1, D), lambda i: (0, 0)),
            pl.BlockSpec((1, D), lambda i: (0, 0)),
            pl.BlockSpec((1, D), lambda i: (0, 0)),
            pl.BlockSpec((1, 1, BLK), lambda i: (i, 0, 0)),
        ],
        out_specs=pl.BlockSpec((G, D), lambda i: (0, 0)),
        out_shape=jax.ShapeDtypeStruct((G, D), jnp.float32),
        scratch_shapes=[
            pltpu.VMEM((G, D), jnp.float32),
            pltpu.VMEM((G, 1), jnp.float32),
        ],
    )(s_, u, deg2, b, g, be, batch3)


def kernel(x, edge_index, batch, Wl0, Wr0, b0, gamma0, beta0,
           Wl1, Wr1, b1, gamma1, beta1, Wl2, Wr2, b2, gamma2, beta2):
    src = edge_index[0]
    dst = edge_index[1]
    pad = E_PAD - E
    src1d = jnp.concatenate([src, jnp.zeros((pad,), jnp.int32)])
    trash = N + (jnp.arange(pad, dtype=jnp.int32) % (NP - N))
    dst1d = jnp.concatenate([dst, trash])
    batch3 = batch.reshape(NBLK, 1, BLK)
    params = [
        (b0.reshape(1, D), gamma0.reshape(1, D), beta0.reshape(1, D), Wl1, Wr1),
        (b1.reshape(1, D), gamma1.reshape(1, D), beta1.reshape(1, D), Wl2, Wr2),
        (b2.reshape(1, D), gamma2.reshape(1, D), beta2.reshape(1, D), None, None),
    ]

    deg2 = _deg(dst1d).reshape(NC, NP, HALF)
    t, u = _mm0(x, Wl0, Wr0)
    for li in range(3):
        s_ = _segsum(t.reshape(NC * NP, HALF), src1d, dst1d).reshape(NC, NP, HALF)
        b, g, be, wl, wr = params[li]
        if li < 2:
            t, u = _mid(s_, u, deg2, b, g, be, wl, wr)
        else:
            return _final(s_, u, deg2, b, g, be, batch3)


# speedup vs baseline: 1.0003x; 1.0003x over previous
import numpy as np
import jax
import jax.numpy as jnp
from jax import lax
from jax.experimental import pallas as pl
from jax.experimental.pallas import tpu as pltpu
from jax.experimental.pallas import tpu_sc as plsc

N = 10000
E = 160000
D = 256
G = 64
EPS = 1e-5

NC = 2
NS = 16
HALF = D // NC
W = 128
WD = 256
WPS = 80
WPD = 20
NWIN_P = NS * WPS
E_PAD = NWIN_P * W
NP = 10240
STRIPE = NP // NS

BLK = 400
NBLK = N // BLK

_mesh = plsc.VectorSubcoreMesh(
    core_axis_name="c", subcore_axis_name="s", num_cores=NC, num_subcores=NS
)

_CHUNKS = [(o, min(128, STRIPE - o)) for o in range(0, STRIPE, 128)]


def _fill_f32(ref, rows, cols, val):
    @pl.loop(0, rows)
    def _(i):
        for j in range(cols // 16):
            ref[i, pl.ds(j * 16, 16)] = jnp.full((16,), val, jnp.float32)


def _wait_dma(hbm_ref, vmem_ref, sem):
    pltpu.make_async_copy(hbm_ref.at[pl.ds(0, W)], vmem_ref, sem).wait()


def _segsum_body(t_hbm, src_hbm, dst_hbm, out_hbm, idx_s, idx_d, rows, acc, sem):
    c = lax.axis_index("c")
    s = lax.axis_index("s")
    base = s * STRIPE

    _fill_f32(rows, W, HALF, 0.0)
    for o, sz in _CHUNKS:
        pltpu.sync_copy(rows.at[pl.ds(0, sz)], acc.at[pl.ds(base + o, sz)])
    plsc.subcore_barrier()

    row_off = c * NP

    @pl.loop(0, WPS)
    def _(i):
        w = s + i * NS
        off = w * W
        pltpu.sync_copy(src_hbm.at[pl.ds(off, W)], idx_s)
        pltpu.sync_copy(dst_hbm.at[pl.ds(off, W)], idx_d)
        for j in range(W // 16):
            sl = pl.ds(j * 16, 16)
            idx_s[sl] = idx_s[sl] + row_off
        pltpu.async_copy(t_hbm.at[idx_s], rows, sem).wait()
        pltpu.sync_copy(rows, acc.at[idx_d], add=True)

    plsc.subcore_barrier()
    for o, sz in _CHUNKS:
        pltpu.sync_copy(
            acc.at[pl.ds(base + o, sz)],
            out_hbm.at[pl.ds(c * NP + base + o, sz)],
        )


_segsum = pl.kernel(
    _segsum_body,
    out_type=jax.ShapeDtypeStruct((NC * NP, HALF), jnp.float32),
    mesh=_mesh,
    scratch_types=[
        pltpu.VMEM((W,), jnp.int32),
        pltpu.VMEM((W,), jnp.int32),
        pltpu.VMEM((W, HALF), jnp.float32),
        pltpu.VMEM_SHARED((NP, HALF), jnp.float32),
        pltpu.SemaphoreType.DMA,
    ],
)


def _deg_body(dst_hbm, out_hbm, idx_d, ones, accd):
    c = lax.axis_index("c")
    s = lax.axis_index("s")
    base = s * STRIPE

    _fill_f32(ones, WD, HALF, 0.0)
    for o, sz in _CHUNKS:
        pltpu.sync_copy(ones.at[pl.ds(0, sz)], accd.at[pl.ds(base + o, sz)])
    _fill_f32(ones, WD, HALF, 1.0)
    plsc.subcore_barrier()

    @pl.loop(0, WPD)
    def _(i):
        off = ((c * NS + s) * WPD + i) * WD
        pltpu.sync_copy(dst_hbm.at[pl.ds(off, WD)], idx_d)
        pltpu.sync_copy(ones, accd.at[idx_d], add=True)

    plsc.subcore_barrier()
    for o, sz in _CHUNKS:
        pltpu.sync_copy(
            accd.at[pl.ds(base + o, sz)],
            out_hbm.at[pl.ds(c * NP + base + o, sz)],
        )


_deg = pl.kernel(
    _deg_body,
    out_type=jax.ShapeDtypeStruct((NC * NP, HALF), jnp.float32),
    mesh=_mesh,
    scratch_types=[
        pltpu.VMEM((WD,), jnp.int32),
        pltpu.VMEM((WD, HALF), jnp.float32),
        pltpu.VMEM_SHARED((NP, HALF), jnp.float32),
    ],
)


def _mm0_body(x_ref, wl_ref, wr_ref, t_ref, u_ref):
    xb = x_ref[...]
    t = jnp.dot(xb, wl_ref[...], preferred_element_type=jnp.float32)
    t_ref[0] = t[:, :HALF]
    t_ref[1] = t[:, HALF:]
    u_ref[...] = jnp.dot(xb, wr_ref[...], preferred_element_type=jnp.float32)


def _mm0(x, wl, wr):
    return pl.pallas_call(
        _mm0_body,
        grid=(NBLK,),
        in_specs=[
            pl.BlockSpec((BLK, D), lambda i: (i, 0)),
            pl.BlockSpec((D, D), lambda i: (0, 0)),
            pl.BlockSpec((D, D), lambda i: (0, 0)),
        ],
        out_specs=[
            pl.BlockSpec((NC, BLK, HALF), lambda i: (0, i, 0)),
            pl.BlockSpec((BLK, D), lambda i: (i, 0)),
        ],
        out_shape=[
            jax.ShapeDtypeStruct((NC, NP, HALF), jnp.float32),
            jax.ShapeDtypeStruct((N, D), jnp.float32),
        ],
    )(x, wl, wr)


def _act(s_ref, u_ref, deg_ref, b_ref, g_ref, be_ref):
    sfull = jnp.concatenate([s_ref[0], s_ref[1]], axis=1)
    deg = deg_ref[0, :, 0:1] + deg_ref[1, :, 0:1]
    agg = sfull / jnp.maximum(deg, 1.0)
    h = agg + u_ref[...] + b_ref[...]
    h = g_ref[...] * (h * (1.0 / np.sqrt(1.0 + EPS))) + be_ref[...]
    return jnp.maximum(h, 0.0)


def _mid_body(s_ref, u_ref, deg_ref, b_ref, g_ref, be_ref, wl_ref, wr_ref,
              t_ref, u2_ref):
    h = _act(s_ref, u_ref, deg_ref, b_ref, g_ref, be_ref)
    t = jnp.dot(h, wl_ref[...], preferred_element_type=jnp.float32)
    t_ref[0] = t[:, :HALF]
    t_ref[1] = t[:, HALF:]
    u2_ref[...] = jnp.dot(h, wr_ref[...], preferred_element_type=jnp.float32)


def _mid(s_, u, deg2, b, g, be, wl, wr):
    return pl.pallas_call(
        _mid_body,
        grid=(NBLK,),
        in_specs=[
            pl.BlockSpec((NC, BLK, HALF), lambda i: (0, i, 0)),
            pl.BlockSpec((BLK, D), lambda i: (i, 0)),
            pl.BlockSpec((NC, BLK, HALF), lambda i: (0, i, 0)),
            pl.BlockSpec((1, D), lambda i: (0, 0)),
            pl.BlockSpec((1, D), lambda i: (0, 0)),
            pl.BlockSpec((1, D), lambda i: (0, 0)),
            pl.BlockSpec((D, D), lambda i: (0, 0)),
            pl.BlockSpec((D, D), lambda i: (0, 0)),
        ],
        out_specs=[
            pl.BlockSpec((NC, BLK, HALF), lambda i: (0, i, 0)),
            pl.BlockSpec((BLK, D), lambda i: (i, 0)),
        ],
        out_shape=[
            jax.ShapeDtypeStruct((NC, NP, HALF), jnp.float32),
            jax.ShapeDtypeStruct((N, D), jnp.float32),
        ],
    )(s_, u, deg2, b, g, be, wl, wr)


def _final_body(s_ref, u_ref, deg_ref, b_ref, g_ref, be_ref, batch_ref,
                out_ref, acc_ref, cnt_ref):
    i = pl.program_id(0)

    @pl.when(i == 0)
    def _():
        acc_ref[...] = jnp.zeros((G, D), jnp.float32)
        cnt_ref[...] = jnp.zeros((G, 1), jnp.float32)

    h = _act(s_ref, u_ref, deg_ref, b_ref, g_ref, be_ref)
    bt = batch_ref[0, 0, :]
    onehot = (bt[None, :] == lax.broadcasted_iota(jnp.int32, (G, BLK), 0))
    onehot = onehot.astype(jnp.float32)
    acc_ref[...] += jnp.dot(onehot, h, preferred_element_type=jnp.float32)
    cnt_ref[...] += jnp.sum(onehot, axis=1, keepdims=True)

    @pl.when(i == NBLK - 1)
    def _():
        out_ref[...] = acc_ref[...] / jnp.maximum(cnt_ref[...], 1.0)


def _final(s_, u, deg2, b, g, be, batch3):
    return pl.pallas_call(
        _final_body,
        grid=(NBLK,),
        in_specs=[
            pl.BlockSpec((NC, BLK, HALF), lambda i: (0, i, 0)),
            pl.BlockSpec((BLK, D), lambda i: (i, 0)),
            pl.BlockSpec((NC, BLK, HALF), lambda i: (0, i, 0)),
            pl.BlockSpec((1, D), lambda i: (0, 0)),
            pl.BlockSpec((1, D), lambda i: (0, 0)),
            pl.BlockSpec((1, D), lambda i: (0, 0)),
            pl.BlockSpec((1, 1, BLK), lambda i: (i, 0, 0)),
        ],
        out_specs=pl.BlockSpec((G, D), lambda i: (0, 0)),
        out_shape=jax.ShapeDtypeStruct((G, D), jnp.float32),
        scratch_shapes=[
            pltpu.VMEM((G, D), jnp.float32),
            pltpu.VMEM((G, 1), jnp.float32),
        ],
    )(s_, u, deg2, b, g, be, batch3)


def kernel(x, edge_index, batch, Wl0, Wr0, b0, gamma0, beta0,
           Wl1, Wr1, b1, gamma1, beta1, Wl2, Wr2, b2, gamma2, beta2):
    src = edge_index[0]
    dst = edge_index[1]
    pad = E_PAD - E
    src1d = jnp.concatenate([src, jnp.zeros((pad,), jnp.int32)])
    trash = N + (jnp.arange(pad, dtype=jnp.int32) % (NP - N))
    dst1d = jnp.concatenate([dst, trash])
    batch3 = batch.reshape(NBLK, 1, BLK)
    params = [
        (b0.reshape(1, D), gamma0.reshape(1, D), beta0.reshape(1, D), Wl1, Wr1),
        (b1.reshape(1, D), gamma1.reshape(1, D), beta1.reshape(1, D), Wl2, Wr2),
        (b2.reshape(1, D), gamma2.reshape(1, D), beta2.reshape(1, D), None, None),
    ]

    deg2 = _deg(dst1d).reshape(NC, NP, HALF)
    t, u = _mm0(x, Wl0, Wr0)
    for li in range(3):
        s_ = _segsum(t.reshape(NC * NP, HALF), src1d, dst1d).reshape(NC, NP, HALF)
        b, g, be, wl, wr = params[li]
        if li < 2:
            t, u = _mid(s_, u, deg2, b, g, be, wl, wr)
        else:
            return _final(s_, u, deg2, b, g, be, batch3)


# speedup vs baseline: 1.4476x; 1.4472x over previous
import numpy as np
import jax
import jax.numpy as jnp
from jax import lax
from jax.experimental import pallas as pl
from jax.experimental.pallas import tpu as pltpu
from jax.experimental.pallas import tpu_sc as plsc

N = 10000
E = 160000
D = 256
G = 64
EPS = 1e-5

NC = 2
NS = 16
HALF = D // NC
W = 128
WD = 256
NWIN = E // W
NWIND = E // WD
NP = 10240
STRIPE = NP // NS

BLK = 400
NBLK = N // BLK

_mesh = plsc.VectorSubcoreMesh(
    core_axis_name="c", subcore_axis_name="s", num_cores=NC, num_subcores=NS
)

_CHUNKS = [(o, min(128, STRIPE - o)) for o in range(0, STRIPE, 128)]


def _fill_f32(ref, rows, cols, val):
    @pl.loop(0, rows)
    def _(i):
        for j in range(cols // 16):
            ref[i, pl.ds(j * 16, 16)] = jnp.full((16,), val, jnp.float32)


def _wait_dma(hbm_ref, vmem_ref, sem):
    pltpu.make_async_copy(hbm_ref.at[pl.ds(0, W)], vmem_ref, sem).wait()


def _segsum_body(t_hbm, src_hbm, dst_hbm, out_hbm, idx_s, idx_d, rows, acc, sem):
    c = lax.axis_index("c")
    s = lax.axis_index("s")
    base = s * STRIPE

    _fill_f32(rows, W, HALF, 0.0)
    for o, sz in _CHUNKS:
        pltpu.sync_copy(rows.at[pl.ds(0, sz)], acc.at[pl.ds(base + o, sz)])
    plsc.subcore_barrier()

    row_off = c * NP

    @pl.loop(0, (NWIN + NS - 1) // NS)
    def _(i):
        w = s + i * NS

        @pl.when(w < NWIN)
        def _():
            off = w * W
            pltpu.sync_copy(src_hbm.at[pl.ds(off, W)], idx_s)
            pltpu.sync_copy(dst_hbm.at[pl.ds(off, W)], idx_d)
            for j in range(W // 16):
                sl = pl.ds(j * 16, 16)
                idx_s[sl] = idx_s[sl] + row_off
            pltpu.async_copy(t_hbm.at[idx_s], rows, sem).wait()
            pltpu.sync_copy(rows, acc.at[idx_d], add=True)

    plsc.subcore_barrier()
    for o, sz in _CHUNKS:
        pltpu.sync_copy(
            acc.at[pl.ds(base + o, sz)],
            out_hbm.at[pl.ds(c * NP + base + o, sz)],
        )


_segsum = pl.kernel(
    _segsum_body,
    out_type=jax.ShapeDtypeStruct((NC * NP, HALF), jnp.float32),
    mesh=_mesh,
    scratch_types=[
        pltpu.VMEM((W,), jnp.int32),
        pltpu.VMEM((W,), jnp.int32),
        pltpu.VMEM((W, HALF), jnp.float32),
        pltpu.VMEM_SHARED((NP, HALF), jnp.float32),
        pltpu.SemaphoreType.DMA,
    ],
)


def _deg_body(dst_hbm, out_hbm, idx_d, ones, accd):
    c = lax.axis_index("c")
    s = lax.axis_index("s")
    base = s * STRIPE

    _fill_f32(ones, WD, HALF, 0.0)
    for o, sz in _CHUNKS:
        pltpu.sync_copy(ones.at[pl.ds(0, sz)], accd.at[pl.ds(base + o, sz)])
    _fill_f32(ones, WD, HALF, 1.0)
    plsc.subcore_barrier()

    wk = c * NS + s

    @pl.loop(0, (NWIND + NC * NS - 1) // (NC * NS))
    def _(i):
        w = wk + i * NC * NS

        @pl.when(w < NWIND)
        def _():
            pltpu.sync_copy(dst_hbm.at[pl.ds(w * WD, WD)], idx_d)
            pltpu.sync_copy(ones, accd.at[idx_d], add=True)

    plsc.subcore_barrier()
    for o, sz in _CHUNKS:
        pltpu.sync_copy(
            accd.at[pl.ds(base + o, sz)],
            out_hbm.at[pl.ds(c * NP + base + o, sz)],
        )


_deg = pl.kernel(
    _deg_body,
    out_type=jax.ShapeDtypeStruct((NC * NP, HALF), jnp.float32),
    mesh=_mesh,
    scratch_types=[
        pltpu.VMEM((WD,), jnp.int32),
        pltpu.VMEM((WD, HALF), jnp.float32),
        pltpu.VMEM_SHARED((NP, HALF), jnp.float32),
    ],
)


def _mm0_body(x_ref, wl_ref, wr_ref, t_ref, u_ref):
    xb = x_ref[...]
    t = jnp.dot(xb, wl_ref[...], preferred_element_type=jnp.float32)
    t_ref[0] = t[:, :HALF]
    t_ref[1] = t[:, HALF:]
    u_ref[...] = jnp.dot(xb, wr_ref[...], preferred_element_type=jnp.float32)


def _mm0(x, wl, wr):
    return pl.pallas_call(
        _mm0_body,
        grid=(NBLK,),
        in_specs=[
            pl.BlockSpec((BLK, D), lambda i: (i, 0)),
            pl.BlockSpec((D, D), lambda i: (0, 0)),
            pl.BlockSpec((D, D), lambda i: (0, 0)),
        ],
        out_specs=[
            pl.BlockSpec((NC, BLK, HALF), lambda i: (0, i, 0)),
            pl.BlockSpec((BLK, D), lambda i: (i, 0)),
        ],
        out_shape=[
            jax.ShapeDtypeStruct((NC, NP, HALF), jnp.float32),
            jax.ShapeDtypeStruct((N, D), jnp.float32),
        ],
    )(x, wl, wr)


def _act(s_ref, u_ref, deg_ref, b_ref, g_ref, be_ref):
    sfull = jnp.concatenate([s_ref[0], s_ref[1]], axis=1)
    deg = deg_ref[0, :, 0:1] + deg_ref[1, :, 0:1]
    agg = sfull / jnp.maximum(deg, 1.0)
    h = agg + u_ref[...] + b_ref[...]
    h = g_ref[...] * (h * (1.0 / np.sqrt(1.0 + EPS))) + be_ref[...]
    return jnp.maximum(h, 0.0)


def _mid_body(s_ref, u_ref, deg_ref, b_ref, g_ref, be_ref, wl_ref, wr_ref,
              t_ref, u2_ref):
    h = _act(s_ref, u_ref, deg_ref, b_ref, g_ref, be_ref)
    t = jnp.dot(h, wl_ref[...], preferred_element_type=jnp.float32)
    t_ref[0] = t[:, :HALF]
    t_ref[1] = t[:, HALF:]
    u2_ref[...] = jnp.dot(h, wr_ref[...], preferred_element_type=jnp.float32)


def _mid(s_, u, deg2, b, g, be, wl, wr):
    return pl.pallas_call(
        _mid_body,
        grid=(NBLK,),
        in_specs=[
            pl.BlockSpec((NC, BLK, HALF), lambda i: (0, i, 0)),
            pl.BlockSpec((BLK, D), lambda i: (i, 0)),
            pl.BlockSpec((NC, BLK, HALF), lambda i: (0, i, 0)),
            pl.BlockSpec((1, D), lambda i: (0, 0)),
            pl.BlockSpec((1, D), lambda i: (0, 0)),
            pl.BlockSpec((1, D), lambda i: (0, 0)),
            pl.BlockSpec((D, D), lambda i: (0, 0)),
            pl.BlockSpec((D, D), lambda i: (0, 0)),
        ],
        out_specs=[
            pl.BlockSpec((NC, BLK, HALF), lambda i: (0, i, 0)),
            pl.BlockSpec((BLK, D), lambda i: (i, 0)),
        ],
        out_shape=[
            jax.ShapeDtypeStruct((NC, NP, HALF), jnp.float32),
            jax.ShapeDtypeStruct((N, D), jnp.float32),
        ],
    )(s_, u, deg2, b, g, be, wl, wr)


def _final_body(s_ref, u_ref, deg_ref, b_ref, g_ref, be_ref, batch_ref,
                out_ref, acc_ref, cnt_ref):
    i = pl.program_id(0)

    @pl.when(i == 0)
    def _():
        acc_ref[...] = jnp.zeros((G, D), jnp.float32)
        cnt_ref[...] = jnp.zeros((G, 1), jnp.float32)

    h = _act(s_ref, u_ref, deg_ref, b_ref, g_ref, be_ref)
    bt = batch_ref[0, 0, :]
    onehot = (bt[None, :] == lax.broadcasted_iota(jnp.int32, (G, BLK), 0))
    onehot = onehot.astype(jnp.float32)
    acc_ref[...] += jnp.dot(onehot, h, preferred_element_type=jnp.float32)
    cnt_ref[...] += jnp.sum(onehot, axis=1, keepdims=True)

    @pl.when(i == NBLK - 1)
    def _():
        out_ref[...] = acc_ref[...] / jnp.maximum(cnt_ref[...], 1.0)


def _final(s_, u, deg2, b, g, be, batch3):
    return pl.pallas_call(
        _final_body,
        grid=(NBLK,),
        in_specs=[
            pl.BlockSpec((NC, BLK, HALF), lambda i: (0, i, 0)),
            pl.BlockSpec((BLK, D), lambda i: (i, 0)),
            pl.BlockSpec((NC, BLK, HALF), lambda i: (0, i, 0)),
            pl.BlockSpec((1, D), lambda i: (0, 0)),
            pl.BlockSpec((1, D), lambda i: (0, 0)),
            pl.BlockSpec((1, D), lambda i: (0, 0)),
            pl.BlockSpec((1, 1, BLK), lambda i: (i, 0, 0)),
        ],
        out_specs=pl.BlockSpec((G, D), lambda i: (0, 0)),
        out_shape=jax.ShapeDtypeStruct((G, D), jnp.float32),
        scratch_shapes=[
            pltpu.VMEM((G, D), jnp.float32),
            pltpu.VMEM((G, 1), jnp.float32),
        ],
    )(s_, u, deg2, b, g, be, batch3)


def kernel(x, edge_index, batch, Wl0, Wr0, b0, gamma0, beta0,
           Wl1, Wr1, b1, gamma1, beta1, Wl2, Wr2, b2, gamma2, beta2):
    src = edge_index[0]
    dst = edge_index[1]
    batch3 = batch.reshape(NBLK, 1, BLK)
    params = [
        (b0.reshape(1, D), gamma0.reshape(1, D), beta0.reshape(1, D), Wl1, Wr1),
        (b1.reshape(1, D), gamma1.reshape(1, D), beta1.reshape(1, D), Wl2, Wr2),
        (b2.reshape(1, D), gamma2.reshape(1, D), beta2.reshape(1, D), None, None),
    ]

    deg2 = _deg(dst).reshape(NC, NP, HALF)
    t, u = _mm0(x, Wl0, Wr0)
    for li in range(3):
        s_ = _segsum(t.reshape(NC * NP, HALF), src, dst).reshape(NC, NP, HALF)
        b, g, be, wl, wr = params[li]
        if li < 2:
            t, u = _mid(s_, u, deg2, b, g, be, wl, wr)
        else:
            return _final(s_, u, deg2, b, g, be, batch3)


# speedup vs baseline: 1.6275x; 1.1242x over previous
import numpy as np
import jax
import jax.numpy as jnp
from jax import lax
from jax.experimental import pallas as pl
from jax.experimental.pallas import tpu as pltpu
from jax.experimental.pallas import tpu_sc as plsc

N = 10000
E = 160000
D = 256
G = 64
EPS = 1e-5

NC = 2
NS = 16
HALF = D // NC
W = 128
WD = 256
NWIN = E // W
NWIND = E // WD
NP = 10240
STRIPE = NP // NS

BLK = 400
NBLK = N // BLK

_mesh = plsc.VectorSubcoreMesh(
    core_axis_name="c", subcore_axis_name="s", num_cores=NC, num_subcores=NS
)

_CHUNKS = [(o, min(128, STRIPE - o)) for o in range(0, STRIPE, 128)]


def _fill_f32(ref, rows, cols, val):
    @pl.loop(0, rows)
    def _(i):
        for j in range(cols // 16):
            ref[i, pl.ds(j * 16, 16)] = jnp.full((16,), val, jnp.float32)


def _wait_dma(hbm_ref, vmem_ref, sem):
    pltpu.make_async_copy(hbm_ref.at[pl.ds(0, W)], vmem_ref, sem).wait()


def _segsum_body(t_hbm, src_hbm, dst_hbm, out_hbm, idx_s, idx_d, rows, acc, sem, sem2, sem3):
    c = lax.axis_index("c")
    s = lax.axis_index("s")
    base = s * STRIPE

    _fill_f32(rows, W, HALF, 0.0)
    for o, sz in _CHUNKS:
        pltpu.sync_copy(rows.at[pl.ds(0, sz)], acc.at[pl.ds(base + o, sz)])
    plsc.subcore_barrier()

    row_off = c * NP

    @pl.loop(0, (NWIN + NS - 1) // NS)
    def _(i):
        w = s + i * NS

        @pl.when(w < NWIN)
        def _():
            off = w * W
            ds = pltpu.async_copy(src_hbm.at[pl.ds(off, W)], idx_s, sem2)
            dd = pltpu.async_copy(dst_hbm.at[pl.ds(off, W)], idx_d, sem3)
            ds.wait()
            dd.wait()
            for j in range(W // 16):
                sl = pl.ds(j * 16, 16)
                idx_s[sl] = idx_s[sl] + row_off
            pltpu.async_copy(t_hbm.at[idx_s], rows, sem).wait()
            pltpu.sync_copy(rows, acc.at[idx_d], add=True)

    plsc.subcore_barrier()
    for o, sz in _CHUNKS:
        pltpu.sync_copy(
            acc.at[pl.ds(base + o, sz)],
            out_hbm.at[pl.ds(c * NP + base + o, sz)],
        )


_segsum = pl.kernel(
    _segsum_body,
    out_type=jax.ShapeDtypeStruct((NC * NP, HALF), jnp.float32),
    mesh=_mesh,
    scratch_types=[
        pltpu.VMEM((W,), jnp.int32),
        pltpu.VMEM((W,), jnp.int32),
        pltpu.VMEM((W, HALF), jnp.float32),
        pltpu.VMEM_SHARED((NP, HALF), jnp.float32),
        pltpu.SemaphoreType.DMA,
        pltpu.SemaphoreType.DMA,
        pltpu.SemaphoreType.DMA,
    ],
)


def _deg_body(dst_hbm, out_hbm, idx_d, ones, accd):
    c = lax.axis_index("c")
    s = lax.axis_index("s")
    base = s * STRIPE

    _fill_f32(ones, WD, HALF, 0.0)
    for o, sz in _CHUNKS:
        pltpu.sync_copy(ones.at[pl.ds(0, sz)], accd.at[pl.ds(base + o, sz)])
    _fill_f32(ones, WD, HALF, 1.0)
    plsc.subcore_barrier()

    wk = c * NS + s

    @pl.loop(0, (NWIND + NC * NS - 1) // (NC * NS))
    def _(i):
        w = wk + i * NC * NS

        @pl.when(w < NWIND)
        def _():
            pltpu.sync_copy(dst_hbm.at[pl.ds(w * WD, WD)], idx_d)
            pltpu.sync_copy(ones, accd.at[idx_d], add=True)

    plsc.subcore_barrier()
    for o, sz in _CHUNKS:
        pltpu.sync_copy(
            accd.at[pl.ds(base + o, sz)],
            out_hbm.at[pl.ds(c * NP + base + o, sz)],
        )


_deg = pl.kernel(
    _deg_body,
    out_type=jax.ShapeDtypeStruct((NC * NP, HALF), jnp.float32),
    mesh=_mesh,
    scratch_types=[
        pltpu.VMEM((WD,), jnp.int32),
        pltpu.VMEM((WD, HALF), jnp.float32),
        pltpu.VMEM_SHARED((NP, HALF), jnp.float32),
    ],
)


def _mm0_body(x_ref, wl_ref, wr_ref, t_ref, u_ref):
    xb = x_ref[...]
    t = jnp.dot(xb, wl_ref[...], preferred_element_type=jnp.float32)
    t_ref[0] = t[:, :HALF]
    t_ref[1] = t[:, HALF:]
    u_ref[...] = jnp.dot(xb, wr_ref[...], preferred_element_type=jnp.float32)


def _mm0(x, wl, wr):
    return pl.pallas_call(
        _mm0_body,
        grid=(NBLK,),
        in_specs=[
            pl.BlockSpec((BLK, D), lambda i: (i, 0)),
            pl.BlockSpec((D, D), lambda i: (0, 0)),
            pl.BlockSpec((D, D), lambda i: (0, 0)),
        ],
        out_specs=[
            pl.BlockSpec((NC, BLK, HALF), lambda i: (0, i, 0)),
            pl.BlockSpec((BLK, D), lambda i: (i, 0)),
        ],
        out_shape=[
            jax.ShapeDtypeStruct((NC, NP, HALF), jnp.float32),
            jax.ShapeDtypeStruct((N, D), jnp.float32),
        ],
    )(x, wl, wr)


def _act(s_ref, u_ref, deg_ref, b_ref, g_ref, be_ref):
    sfull = jnp.concatenate([s_ref[0], s_ref[1]], axis=1)
    deg = deg_ref[0, :, 0:1] + deg_ref[1, :, 0:1]
    agg = sfull / jnp.maximum(deg, 1.0)
    h = agg + u_ref[...] + b_ref[...]
    h = g_ref[...] * (h * (1.0 / np.sqrt(1.0 + EPS))) + be_ref[...]
    return jnp.maximum(h, 0.0)


def _mid_body(s_ref, u_ref, deg_ref, b_ref, g_ref, be_ref, wl_ref, wr_ref,
              t_ref, u2_ref):
    h = _act(s_ref, u_ref, deg_ref, b_ref, g_ref, be_ref)
    t = jnp.dot(h, wl_ref[...], preferred_element_type=jnp.float32)
    t_ref[0] = t[:, :HALF]
    t_ref[1] = t[:, HALF:]
    u2_ref[...] = jnp.dot(h, wr_ref[...], preferred_element_type=jnp.float32)


def _mid(s_, u, deg2, b, g, be, wl, wr):
    return pl.pallas_call(
        _mid_body,
        grid=(NBLK,),
        in_specs=[
            pl.BlockSpec((NC, BLK, HALF), lambda i: (0, i, 0)),
            pl.BlockSpec((BLK, D), lambda i: (i, 0)),
            pl.BlockSpec((NC, BLK, HALF), lambda i: (0, i, 0)),
            pl.BlockSpec((1, D), lambda i: (0, 0)),
            pl.BlockSpec((1, D), lambda i: (0, 0)),
            pl.BlockSpec((1, D), lambda i: (0, 0)),
            pl.BlockSpec((D, D), lambda i: (0, 0)),
            pl.BlockSpec((D, D), lambda i: (0, 0)),
        ],
        out_specs=[
            pl.BlockSpec((NC, BLK, HALF), lambda i: (0, i, 0)),
            pl.BlockSpec((BLK, D), lambda i: (i, 0)),
        ],
        out_shape=[
            jax.ShapeDtypeStruct((NC, NP, HALF), jnp.float32),
            jax.ShapeDtypeStruct((N, D), jnp.float32),
        ],
    )(s_, u, deg2, b, g, be, wl, wr)


def _final_body(s_ref, u_ref, deg_ref, b_ref, g_ref, be_ref, batch_ref,
                out_ref, acc_ref, cnt_ref):
    i = pl.program_id(0)

    @pl.when(i == 0)
    def _():
        acc_ref[...] = jnp.zeros((G, D), jnp.float32)
        cnt_ref[...] = jnp.zeros((G, 1), jnp.float32)

    h = _act(s_ref, u_ref, deg_ref, b_ref, g_ref, be_ref)
    bt = batch_ref[0, 0, :]
    onehot = (bt[None, :] == lax.broadcasted_iota(jnp.int32, (G, BLK), 0))
    onehot = onehot.astype(jnp.float32)
    acc_ref[...] += jnp.dot(onehot, h, preferred_element_type=jnp.float32)
    cnt_ref[...] += jnp.sum(onehot, axis=1, keepdims=True)

    @pl.when(i == NBLK - 1)
    def _():
        out_ref[...] = acc_ref[...] / jnp.maximum(cnt_ref[...], 1.0)


def _final(s_, u, deg2, b, g, be, batch3):
    return pl.pallas_call(
        _final_body,
        grid=(NBLK,),
        in_specs=[
            pl.BlockSpec((NC, BLK, HALF), lambda i: (0, i, 0)),
            pl.BlockSpec((BLK, D), lambda i: (i, 0)),
            pl.BlockSpec((NC, BLK, HALF), lambda i: (0, i, 0)),
            pl.BlockSpec((1, D), lambda i: (0, 0)),
            pl.BlockSpec((1, D), lambda i: (0, 0)),
            pl.BlockSpec((1, D), lambda i: (0, 0)),
            pl.BlockSpec((1, 1, BLK), lambda i: (i, 0, 0)),
        ],
        out_specs=pl.BlockSpec((G, D), lambda i: (0, 0)),
        out_shape=jax.ShapeDtypeStruct((G, D), jnp.float32),
        scratch_shapes=[
            pltpu.VMEM((G, D), jnp.float32),
            pltpu.VMEM((G, 1), jnp.float32),
        ],
    )(s_, u, deg2, b, g, be, batch3)


def kernel(x, edge_index, batch, Wl0, Wr0, b0, gamma0, beta0,
           Wl1, Wr1, b1, gamma1, beta1, Wl2, Wr2, b2, gamma2, beta2):
    src = edge_index[0]
    dst = edge_index[1]
    batch3 = batch.reshape(NBLK, 1, BLK)
    params = [
        (b0.reshape(1, D), gamma0.reshape(1, D), beta0.reshape(1, D), Wl1, Wr1),
        (b1.reshape(1, D), gamma1.reshape(1, D), beta1.reshape(1, D), Wl2, Wr2),
        (b2.reshape(1, D), gamma2.reshape(1, D), beta2.reshape(1, D), None, None),
    ]

    deg2 = _deg(dst).reshape(NC, NP, HALF)
    t, u = _mm0(x, Wl0, Wr0)
    for li in range(3):
        s_ = _segsum(t.reshape(NC * NP, HALF), src, dst).reshape(NC, NP, HALF)
        b, g, be, wl, wr = params[li]
        if li < 2:
            t, u = _mid(s_, u, deg2, b, g, be, wl, wr)
        else:
            return _final(s_, u, deg2, b, g, be, batch3)


# speedup vs baseline: 1.9425x; 1.1936x over previous
import numpy as np
import jax
import jax.numpy as jnp
from jax import lax
from jax.experimental import pallas as pl
from jax.experimental.pallas import tpu as pltpu
from jax.experimental.pallas import tpu_sc as plsc

N = 10000
E = 160000
D = 256
G = 64
EPS = 1e-5

NC = 2
NS = 16
HALF = D // NC
W = 128
WD = 256
NWIN = E // W
NWIND = E // WD
NP = 10240
STRIPE = NP // NS

BLK = 400
NBLK = N // BLK

_mesh = plsc.VectorSubcoreMesh(
    core_axis_name="c", subcore_axis_name="s", num_cores=NC, num_subcores=NS
)

_CHUNKS = [(o, min(128, STRIPE - o)) for o in range(0, STRIPE, 128)]


def _fill_f32(ref, rows, cols, val):
    @pl.loop(0, rows)
    def _(i):
        for j in range(cols // 16):
            ref[i, pl.ds(j * 16, 16)] = jnp.full((16,), val, jnp.float32)


def _wait_dma(hbm_ref, vmem_ref, sem):
    pltpu.make_async_copy(hbm_ref.at[pl.ds(0, W)], vmem_ref, sem).wait()


def _segsum_body(t_hbm, src_hbm, dst_hbm, out_hbm, idx_s, idx_d, idx_s1,
                 idx_d1, rows, rows1, acc, g0, g1, a0, a1, a2, a3):
    c = lax.axis_index("c")
    s = lax.axis_index("s")
    base = s * STRIPE

    _fill_f32(rows, W, HALF, 0.0)
    for o, sz in _CHUNKS:
        pltpu.sync_copy(rows.at[pl.ds(0, sz)], acc.at[pl.ds(base + o, sz)])
    plsc.subcore_barrier()

    row_off = c * NP

    def adjust(ref):
        for j in range(W // 16):
            sl = pl.ds(j * 16, 16)
            ref[sl] = ref[sl] + row_off

    @pl.loop(0, NWIN // (2 * NS))
    def _(i):
        w0 = s + 2 * NS * i
        w1 = w0 + NS
        ds0 = pltpu.async_copy(src_hbm.at[pl.ds(w0 * W, W)], idx_s, a0)
        dd0 = pltpu.async_copy(dst_hbm.at[pl.ds(w0 * W, W)], idx_d, a1)
        ds1 = pltpu.async_copy(src_hbm.at[pl.ds(w1 * W, W)], idx_s1, a2)
        dd1 = pltpu.async_copy(dst_hbm.at[pl.ds(w1 * W, W)], idx_d1, a3)
        ds0.wait()
        adjust(idx_s)
        gd0 = pltpu.async_copy(t_hbm.at[idx_s], rows, g0)
        ds1.wait()
        adjust(idx_s1)
        gd0.wait()
        gd1 = pltpu.async_copy(t_hbm.at[idx_s1], rows1, g1)
        dd0.wait()
        pltpu.sync_copy(rows, acc.at[idx_d], add=True)
        gd1.wait()
        dd1.wait()
        pltpu.sync_copy(rows1, acc.at[idx_d1], add=True)

    for w in ((NWIN // (2 * NS)) * 2 * NS + s,):
        @pl.when(w < NWIN)
        def _():
            ds0 = pltpu.async_copy(src_hbm.at[pl.ds(w * W, W)], idx_s, a0)
            dd0 = pltpu.async_copy(dst_hbm.at[pl.ds(w * W, W)], idx_d, a1)
            ds0.wait()
            dd0.wait()
            adjust(idx_s)
            pltpu.async_copy(t_hbm.at[idx_s], rows, g0).wait()
            pltpu.sync_copy(rows, acc.at[idx_d], add=True)

    plsc.subcore_barrier()
    for o, sz in _CHUNKS:
        pltpu.sync_copy(
            acc.at[pl.ds(base + o, sz)],
            out_hbm.at[pl.ds(c * NP + base + o, sz)],
        )


_segsum = pl.kernel(
    _segsum_body,
    out_type=jax.ShapeDtypeStruct((NC * NP, HALF), jnp.float32),
    mesh=_mesh,
    scratch_types=[
        pltpu.VMEM((W,), jnp.int32),
        pltpu.VMEM((W,), jnp.int32),
        pltpu.VMEM((W,), jnp.int32),
        pltpu.VMEM((W,), jnp.int32),
        pltpu.VMEM((W, HALF), jnp.float32),
        pltpu.VMEM((W, HALF), jnp.float32),
        pltpu.VMEM_SHARED((NP, HALF), jnp.float32),
        pltpu.SemaphoreType.DMA,
        pltpu.SemaphoreType.DMA,
        pltpu.SemaphoreType.DMA,
        pltpu.SemaphoreType.DMA,
        pltpu.SemaphoreType.DMA,
        pltpu.SemaphoreType.DMA,
    ],
)


def _deg_body(dst_hbm, out_hbm, idx_d, ones, accd):
    c = lax.axis_index("c")
    s = lax.axis_index("s")
    base = s * STRIPE

    _fill_f32(ones, WD, HALF, 0.0)
    for o, sz in _CHUNKS:
        pltpu.sync_copy(ones.at[pl.ds(0, sz)], accd.at[pl.ds(base + o, sz)])
    _fill_f32(ones, WD, HALF, 1.0)
    plsc.subcore_barrier()

    wk = c * NS + s

    @pl.loop(0, (NWIND + NC * NS - 1) // (NC * NS))
    def _(i):
        w = wk + i * NC * NS

        @pl.when(w < NWIND)
        def _():
            pltpu.sync_copy(dst_hbm.at[pl.ds(w * WD, WD)], idx_d)
            pltpu.sync_copy(ones, accd.at[idx_d], add=True)

    plsc.subcore_barrier()
    for o, sz in _CHUNKS:
        pltpu.sync_copy(
            accd.at[pl.ds(base + o, sz)],
            out_hbm.at[pl.ds(c * NP + base + o, sz)],
        )


_deg = pl.kernel(
    _deg_body,
    out_type=jax.ShapeDtypeStruct((NC * NP, HALF), jnp.float32),
    mesh=_mesh,
    scratch_types=[
        pltpu.VMEM((WD,), jnp.int32),
        pltpu.VMEM((WD, HALF), jnp.float32),
        pltpu.VMEM_SHARED((NP, HALF), jnp.float32),
    ],
)


def _mm0_body(x_ref, wl_ref, wr_ref, t_ref, u_ref):
    xb = x_ref[...]
    t = jnp.dot(xb, wl_ref[...], preferred_element_type=jnp.float32)
    t_ref[0] = t[:, :HALF]
    t_ref[1] = t[:, HALF:]
    u_ref[...] = jnp.dot(xb, wr_ref[...], preferred_element_type=jnp.float32)


def _mm0(x, wl, wr):
    return pl.pallas_call(
        _mm0_body,
        grid=(NBLK,),
        in_specs=[
            pl.BlockSpec((BLK, D), lambda i: (i, 0)),
            pl.BlockSpec((D, D), lambda i: (0, 0)),
            pl.BlockSpec((D, D), lambda i: (0, 0)),
        ],
        out_specs=[
            pl.BlockSpec((NC, BLK, HALF), lambda i: (0, i, 0)),
            pl.BlockSpec((BLK, D), lambda i: (i, 0)),
        ],
        out_shape=[
            jax.ShapeDtypeStruct((NC, NP, HALF), jnp.float32),
            jax.ShapeDtypeStruct((N, D), jnp.float32),
        ],
    )(x, wl, wr)


def _act(s_ref, u_ref, deg_ref, b_ref, g_ref, be_ref):
    sfull = jnp.concatenate([s_ref[0], s_ref[1]], axis=1)
    deg = deg_ref[0, :, 0:1] + deg_ref[1, :, 0:1]
    agg = sfull / jnp.maximum(deg, 1.0)
    h = agg + u_ref[...] + b_ref[...]
    h = g_ref[...] * (h * (1.0 / np.sqrt(1.0 + EPS))) + be_ref[...]
    return jnp.maximum(h, 0.0)


def _mid_body(s_ref, u_ref, deg_ref, b_ref, g_ref, be_ref, wl_ref, wr_ref,
              t_ref, u2_ref):
    h = _act(s_ref, u_ref, deg_ref, b_ref, g_ref, be_ref)
    t = jnp.dot(h, wl_ref[...], preferred_element_type=jnp.float32)
    t_ref[0] = t[:, :HALF]
    t_ref[1] = t[:, HALF:]
    u2_ref[...] = jnp.dot(h, wr_ref[...], preferred_element_type=jnp.float32)


def _mid(s_, u, deg2, b, g, be, wl, wr):
    return pl.pallas_call(
        _mid_body,
        grid=(NBLK,),
        in_specs=[
            pl.BlockSpec((NC, BLK, HALF), lambda i: (0, i, 0)),
            pl.BlockSpec((BLK, D), lambda i: (i, 0)),
            pl.BlockSpec((NC, BLK, HALF), lambda i: (0, i, 0)),
            pl.BlockSpec((1, D), lambda i: (0, 0)),
            pl.BlockSpec((1, D), lambda i: (0, 0)),
            pl.BlockSpec((1, D), lambda i: (0, 0)),
            pl.BlockSpec((D, D), lambda i: (0, 0)),
            pl.BlockSpec((D, D), lambda i: (0, 0)),
        ],
        out_specs=[
            pl.BlockSpec((NC, BLK, HALF), lambda i: (0, i, 0)),
            pl.BlockSpec((BLK, D), lambda i: (i, 0)),
        ],
        out_shape=[
            jax.ShapeDtypeStruct((NC, NP, HALF), jnp.float32),
            jax.ShapeDtypeStruct((N, D), jnp.float32),
        ],
    )(s_, u, deg2, b, g, be, wl, wr)


def _final_body(s_ref, u_ref, deg_ref, b_ref, g_ref, be_ref, batch_ref,
                out_ref, acc_ref, cnt_ref):
    i = pl.program_id(0)

    @pl.when(i == 0)
    def _():
        acc_ref[...] = jnp.zeros((G, D), jnp.float32)
        cnt_ref[...] = jnp.zeros((G, 1), jnp.float32)

    h = _act(s_ref, u_ref, deg_ref, b_ref, g_ref, be_ref)
    bt = batch_ref[0, 0, :]
    onehot = (bt[None, :] == lax.broadcasted_iota(jnp.int32, (G, BLK), 0))
    onehot = onehot.astype(jnp.float32)
    acc_ref[...] += jnp.dot(onehot, h, preferred_element_type=jnp.float32)
    cnt_ref[...] += jnp.sum(onehot, axis=1, keepdims=True)

    @pl.when(i == NBLK - 1)
    def _():
        out_ref[...] = acc_ref[...] / jnp.maximum(cnt_ref[...], 1.0)


def _final(s_, u, deg2, b, g, be, batch3):
    return pl.pallas_call(
        _final_body,
        grid=(NBLK,),
        in_specs=[
            pl.BlockSpec((NC, BLK, HALF), lambda i: (0, i, 0)),
            pl.BlockSpec((BLK, D), lambda i: (i, 0)),
            pl.BlockSpec((NC, BLK, HALF), lambda i: (0, i, 0)),
            pl.BlockSpec((1, D), lambda i: (0, 0)),
            pl.BlockSpec((1, D), lambda i: (0, 0)),
            pl.BlockSpec((1, D), lambda i: (0, 0)),
            pl.BlockSpec((1, 1, BLK), lambda i: (i, 0, 0)),
        ],
        out_specs=pl.BlockSpec((G, D), lambda i: (0, 0)),
        out_shape=jax.ShapeDtypeStruct((G, D), jnp.float32),
        scratch_shapes=[
            pltpu.VMEM((G, D), jnp.float32),
            pltpu.VMEM((G, 1), jnp.float32),
        ],
    )(s_, u, deg2, b, g, be, batch3)


def kernel(x, edge_index, batch, Wl0, Wr0, b0, gamma0, beta0,
           Wl1, Wr1, b1, gamma1, beta1, Wl2, Wr2, b2, gamma2, beta2):
    src = edge_index[0]
    dst = edge_index[1]
    batch3 = batch.reshape(NBLK, 1, BLK)
    params = [
        (b0.reshape(1, D), gamma0.reshape(1, D), beta0.reshape(1, D), Wl1, Wr1),
        (b1.reshape(1, D), gamma1.reshape(1, D), beta1.reshape(1, D), Wl2, Wr2),
        (b2.reshape(1, D), gamma2.reshape(1, D), beta2.reshape(1, D), None, None),
    ]

    deg2 = _deg(dst).reshape(NC, NP, HALF)
    t, u = _mm0(x, Wl0, Wr0)
    for li in range(3):
        s_ = _segsum(t.reshape(NC * NP, HALF), src, dst).reshape(NC, NP, HALF)
        b, g, be, wl, wr = params[li]
        if li < 2:
            t, u = _mid(s_, u, deg2, b, g, be, wl, wr)
        else:
            return _final(s_, u, deg2, b, g, be, batch3)
